# baseline (speedup 1.0000x reference)
import functools

import jax
import jax.numpy as jnp
from jax import lax
from jax.experimental import pallas as pl
from jax.experimental.pallas import tpu as pltpu
from jax.experimental.pallas import tpu_sc as plsc

NUM_CORES = 2
NUM_SUBCORES = 16
NUM_WORKERS = NUM_CORES * NUM_SUBCORES


def _gather_kernel(b0, b1, d):
    rows_per_w = b0 // NUM_WORKERS
    mesh = plsc.VectorSubcoreMesh(core_axis_name="c", subcore_axis_name="s")

    assert rows_per_w % 2 == 0

    @functools.partial(
        pl.kernel,
        mesh=mesh,
        out_type=jax.ShapeDtypeStruct((b0, b1, d), jnp.float32),
        scratch_types=[
            pltpu.VMEM((b1,), jnp.int32),
            pltpu.VMEM((b1,), jnp.int32),
            pltpu.VMEM((b1, d), jnp.float32),
            pltpu.VMEM((b1, d), jnp.float32),
            pltpu.SemaphoreType.DMA,
            pltpu.SemaphoreType.DMA,
            pltpu.SemaphoreType.DMA,
            pltpu.SemaphoreType.DMA,
        ],
        compiler_params=pltpu.CompilerParams(use_tc_tiling_on_sc=False),
    )
    def k(x_hbm, table_hbm, out_hbm, idx_v0, idx_v1, rows_v0, rows_v1,
          gsem0, gsem1, wsem0, wsem1):
        wid = lax.axis_index("s") * NUM_CORES + lax.axis_index("c")
        base = wid * rows_per_w
        idx_v = (idx_v0, idx_v1)
        rows_v = (rows_v0, rows_v1)
        gsem = (gsem0, gsem1)
        wsem = (wsem0, wsem1)

        pltpu.sync_copy(x_hbm.at[base], idx_v[0])
        pltpu.async_copy(table_hbm.at[idx_v[0]], rows_v[0], gsem[0])

        def body(gi, carry):
            c0 = gi * 2
            for p in (0, 1):
                c = c0 + p
                q = p ^ 1
                @pl.when(c + 1 < rows_per_w)
                def _():
                    @pl.when(c + 1 >= 2)
                    def _():
                        pltpu.make_async_copy(
                            rows_v[q], out_hbm.at[base + c + 1], wsem[q]
                        ).wait()
                    pltpu.sync_copy(x_hbm.at[base + c + 1], idx_v[q])
                    pltpu.async_copy(table_hbm.at[idx_v[q]], rows_v[q], gsem[q])
                pltpu.make_async_copy(
                    table_hbm.at[idx_v[p]], rows_v[p], gsem[p]
                ).wait()
                pltpu.async_copy(rows_v[p], out_hbm.at[base + c], wsem[p])
            return carry

        lax.fori_loop(0, rows_per_w // 2, body, 0)
        pltpu.make_async_copy(
            rows_v[0], out_hbm.at[base], wsem[0]
        ).wait()
        pltpu.make_async_copy(
            rows_v[1], out_hbm.at[base + rows_per_w - 1], wsem[1]
        ).wait()

    return k


def kernel(x, table):
    b0, b1 = x.shape
    return _gather_kernel(b0, b1, table.shape[1])(x, table)
